# baseline (speedup 1.0000x reference)
import functools
import math

import jax
import jax.numpy as jnp
from jax import lax
from jax.experimental import pallas as pl
from jax.experimental.pallas import tpu as pltpu
from jax.experimental.pallas import tpu_sc as plsc

_CHUNK = 128


@functools.partial(jax.jit, static_argnames=("n_workers", "n_chunks"))
def _emb_lookup(table, idx, *, n_workers, n_chunks):
    vocab, d = table.shape
    b_per_w = n_chunks * _CHUNK
    scale = math.sqrt(d)
    mesh = plsc.VectorSubcoreMesh(core_axis_name="c", subcore_axis_name="s")
    n_cores = plsc.get_sparse_core_info().num_cores

    @functools.partial(
        pl.kernel,
        mesh=mesh,
        out_type=jax.ShapeDtypeStruct((n_workers * b_per_w, d), jnp.float32),
        scratch_types=[
            pltpu.VMEM((n_chunks, _CHUNK), jnp.int32),
            pltpu.VMEM((_CHUNK, d), jnp.float32),
            pltpu.SemaphoreType.DMA,
        ],
        compiler_params=pltpu.CompilerParams(use_tc_tiling_on_sc=False),
    )
    def k(table_hbm, idx_hbm, out_hbm, idx_v, rows_v, sem):
        wid = lax.axis_index("s") * n_cores + lax.axis_index("c")
        base = wid * b_per_w
        pltpu.sync_copy(idx_hbm.at[wid], idx_v)

        def chunk_body(j, carry):
            pltpu.async_copy(table_hbm.at[idx_v.at[j]], rows_v, sem).wait()

            def scale_body(r, c):
                for kk in range(d // 16):
                    sl = pl.ds(kk * 16, 16)
                    rows_v[r, sl] = rows_v[r, sl] * scale
                return c

            lax.fori_loop(0, _CHUNK, scale_body, 0)
            pltpu.sync_copy(rows_v, out_hbm.at[pl.ds(base + j * _CHUNK, _CHUNK)])
            return carry

        lax.fori_loop(0, n_chunks, chunk_body, 0)

    return k(table, idx)


def kernel(tokens, table):
    b0, b1 = tokens.shape
    d = table.shape[1]
    n_workers = 32
    total = b0 * b1
    n_chunks = total // (n_workers * _CHUNK)
    idx = tokens.astype(jnp.int32).reshape(n_workers, n_chunks, _CHUNK)
    out = _emb_lookup(table, idx, n_workers=n_workers, n_chunks=n_chunks)
    return out.reshape(b0, b1, d)


# speedup vs baseline: 1.1418x; 1.1418x over previous
import functools
import math

import jax
import jax.numpy as jnp
from jax import lax
from jax.experimental import pallas as pl
from jax.experimental.pallas import tpu as pltpu
from jax.experimental.pallas import tpu_sc as plsc

_CHUNK = 128


@functools.partial(jax.jit, static_argnames=("n_workers", "n_chunks"))
def _emb_lookup(table, idx, *, n_workers, n_chunks):
    vocab, d = table.shape
    b_per_w = n_chunks * _CHUNK
    mesh = plsc.VectorSubcoreMesh(core_axis_name="c", subcore_axis_name="s")
    n_cores = plsc.get_sparse_core_info().num_cores
    scale = math.sqrt(d)

    @functools.partial(
        pl.kernel,
        mesh=mesh,
        out_type=jax.ShapeDtypeStruct((n_workers * b_per_w, d), jnp.float32),
        scratch_types=[
            pltpu.VMEM((n_chunks, _CHUNK), jnp.int32),
            pltpu.VMEM((_CHUNK, d), jnp.float32),
            pltpu.VMEM((_CHUNK, d), jnp.float32),
            pltpu.SemaphoreType.DMA,
            pltpu.SemaphoreType.DMA,
            pltpu.SemaphoreType.DMA,
            pltpu.SemaphoreType.DMA,
        ],
        compiler_params=pltpu.CompilerParams(use_tc_tiling_on_sc=False),
    )
    def k(table_hbm, idx_hbm, out_hbm, idx_v, rows_a, rows_b, ga, gb, sa, sb):
        wid = lax.axis_index("s") * n_cores + lax.axis_index("c")
        base = wid * b_per_w
        pltpu.sync_copy(idx_hbm.at[wid], idx_v)

        def gather(j, buf, sem):
            pltpu.async_copy(table_hbm.at[idx_v.at[j]], buf, sem)

        def scale_rows(buf):
            def scale_body(r, c):
                for kk in range(d // 16):
                    sl = pl.ds(kk * 16, 16)
                    buf[r, sl] = buf[r, sl] * scale
                return c

            lax.fori_loop(0, _CHUNK, scale_body, 0)

        def gather_wait(buf, sem):
            pltpu.make_async_copy(table_hbm.at[idx_v.at[0]], buf, sem).wait()

        def scatter(j, buf, sem):
            pltpu.async_copy(buf, out_hbm.at[pl.ds(base + j * _CHUNK, _CHUNK)], sem)

        def scatter_wait(buf, sem):
            pltpu.make_async_copy(
                buf, out_hbm.at[pl.ds(base, _CHUNK)], sem
            ).wait()

        gather(0, rows_a, ga)

        def body(p, carry):
            j0 = 2 * p

            @pl.when(p > 0)
            def _():
                scatter_wait(rows_b, sb)

            gather(j0 + 1, rows_b, gb)
            gather_wait(rows_a, ga)
            scale_rows(rows_a)
            scatter(j0, rows_a, sa)

            @pl.when(p < n_chunks // 2 - 1)
            def _():
                scatter_wait(rows_a, sa)
                gather(j0 + 2, rows_a, ga)

            gather_wait(rows_b, gb)
            scale_rows(rows_b)
            scatter(j0 + 1, rows_b, sb)
            return carry

        lax.fori_loop(0, n_chunks // 2, body, 0)
        scatter_wait(rows_a, sa)
        scatter_wait(rows_b, sb)

    return k(table, idx)


def kernel(tokens, table):
    b0, b1 = tokens.shape
    d = table.shape[1]
    n_workers = 32
    total = b0 * b1
    n_chunks = total // (n_workers * _CHUNK)
    idx = tokens.astype(jnp.int32).reshape(n_workers, n_chunks, _CHUNK)
    out = _emb_lookup(table, idx, n_workers=n_workers, n_chunks=n_chunks)
    return out.reshape(b0, b1, d)


# speedup vs baseline: 1.3185x; 1.1547x over previous
import functools
import math

import jax
import jax.numpy as jnp
from jax import lax
from jax.experimental import pallas as pl
from jax.experimental.pallas import tpu as pltpu
from jax.experimental.pallas import tpu_sc as plsc

_LB = 128
_TP = 129


@functools.partial(jax.jit, static_argnames=("n_workers", "n_seq"))
def _emb_lookup(table, idx, *, n_workers, n_seq):
    vocab, d = table.shape
    scale = math.sqrt(d)
    mesh = plsc.VectorSubcoreMesh(core_axis_name="c", subcore_axis_name="s")
    n_cores = plsc.get_sparse_core_info().num_cores
    d_hi = d // 8

    @functools.partial(
        pl.kernel,
        mesh=mesh,
        out_type=jax.ShapeDtypeStruct((n_seq, d_hi, n_workers, 8, _LB), jnp.float32),
        scratch_types=[
            pltpu.VMEM((n_seq, _LB), jnp.int32),
            pltpu.VMEM((_LB, d), jnp.float32),
            pltpu.VMEM((_LB, d), jnp.float32),
            pltpu.VMEM((d, _TP), jnp.float32),
            pltpu.VMEM((d, _TP), jnp.float32),
            pltpu.SemaphoreType.DMA,
            pltpu.SemaphoreType.DMA,
            pltpu.SemaphoreType.DMA,
            pltpu.SemaphoreType.DMA,
        ],
        compiler_params=pltpu.CompilerParams(
            use_tc_tiling_on_sc=False, needs_layout_passes=False
        ),
    )
    def k(table_hbm, idx_hbm, out_hbm, idx_v, g_a, g_b, t_a, t_b, ga, gb, sa, sb):
        wid = lax.axis_index("s") * n_cores + lax.axis_index("c")
        pltpu.sync_copy(idx_hbm.at[wid], idx_v)

        lanes = lax.iota(jnp.int32, 16)
        zeros = jnp.zeros((16,), jnp.int32)

        def gather(s, buf, sem):
            pltpu.async_copy(table_hbm.at[idx_v.at[s]], buf, sem)

        def gather_wait(buf, sem):
            pltpu.make_async_copy(table_hbm.at[idx_v.at[0]], buf, sem).wait()

        def transpose_scale(gbuf, tbuf):
            def rbody(r, c):
                col = zeros + r
                for kk in range(d // 16):
                    erow = lanes + 16 * kk
                    v = gbuf[r, pl.ds(16 * kk, 16)]
                    plsc.store_scatter(tbuf, [erow, col], v * scale)
                return c

            lax.fori_loop(0, _LB, rbody, 0)

        def store(s, tbuf, sem):
            for eh in range(d_hi):
                pltpu.async_copy(
                    tbuf.at[pl.ds(8 * eh, 8), pl.ds(0, _LB)],
                    out_hbm.at[s, eh, wid],
                    sem,
                )

        def store_wait(tbuf, sem):
            for eh in range(d_hi):
                pltpu.make_async_copy(
                    tbuf.at[pl.ds(8 * eh, 8), pl.ds(0, _LB)],
                    out_hbm.at[0, eh, wid],
                    sem,
                ).wait()

        gather(0, g_a, ga)

        def body(p, carry):
            s0 = 2 * p

            @pl.when(p > 0)
            def _():
                store_wait(t_a, sa)

            gather(s0 + 1, g_b, gb)
            gather_wait(g_a, ga)
            transpose_scale(g_a, t_a)
            store(s0, t_a, sa)

            @pl.when(p < n_seq // 2 - 1)
            def _():
                gather(s0 + 2, g_a, ga)

            @pl.when(p > 0)
            def _():
                store_wait(t_b, sb)

            gather_wait(g_b, gb)
            transpose_scale(g_b, t_b)
            store(s0 + 1, t_b, sb)
            return carry

        lax.fori_loop(0, n_seq // 2, body, 0)
        store_wait(t_a, sa)
        store_wait(t_b, sb)

    return k(table, idx)


def kernel(tokens, table):
    b0, b1 = tokens.shape
    d = table.shape[1]
    n_workers = b0 // _LB
    idx = tokens.T.reshape(b1, n_workers, _LB).transpose(1, 0, 2).astype(jnp.int32)
    out5 = _emb_lookup(table, idx, n_workers=n_workers, n_seq=b1)
    return out5.transpose(2, 4, 0, 1, 3).reshape(b0, b1, d)


# speedup vs baseline: 1.6135x; 1.2238x over previous
import functools
import math

import jax
import jax.numpy as jnp
from jax import lax
from jax.experimental import pallas as pl
from jax.experimental.pallas import tpu as pltpu
from jax.experimental.pallas import tpu_sc as plsc

_LB = 128
_TP = 129
_CB = 2048


def _detile_body(x_ref, o_ref):
    x = x_ref[...]
    h = _CB // 2
    x2 = jnp.concatenate([x[:, :h], x[:, h:]], axis=0)
    o_ref[...] = jnp.transpose(x2)


def _detile(table_t):
    emb, vocab = table_t.shape
    grid = (vocab + _CB - 1) // _CB
    return pl.pallas_call(
        _detile_body,
        grid=(grid,),
        in_specs=[pl.BlockSpec((emb, _CB), lambda i: (0, i))],
        out_specs=pl.BlockSpec((_CB // 2, 2 * emb), lambda i: (i, 0)),
        out_shape=jax.ShapeDtypeStruct((grid * _CB // 2, 2 * emb), jnp.float32),
    )(table_t)


@functools.partial(jax.jit, static_argnames=("n_workers", "n_seq"))
def _emb_lookup(table, idx, *, n_workers, n_seq):
    vocab, d = table.shape
    scale = math.sqrt(d)
    mesh = plsc.VectorSubcoreMesh(core_axis_name="c", subcore_axis_name="s")
    n_cores = plsc.get_sparse_core_info().num_cores
    d_hi = d // 8

    @functools.partial(
        pl.kernel,
        mesh=mesh,
        out_type=jax.ShapeDtypeStruct((n_seq, d_hi, n_workers, 8, _LB), jnp.float32),
        scratch_types=[
            pltpu.VMEM((n_seq, _LB), jnp.int32),
            pltpu.VMEM((_LB, d), jnp.float32),
            pltpu.VMEM((_LB, d), jnp.float32),
            pltpu.VMEM((d, _TP), jnp.float32),
            pltpu.VMEM((d, _TP), jnp.float32),
            pltpu.SemaphoreType.DMA,
            pltpu.SemaphoreType.DMA,
            pltpu.SemaphoreType.DMA,
            pltpu.SemaphoreType.DMA,
        ],
        compiler_params=pltpu.CompilerParams(
            use_tc_tiling_on_sc=False, needs_layout_passes=False
        ),
    )
    def k(table_hbm, idx_hbm, out_hbm, idx_v, g_a, g_b, t_a, t_b, ga, gb, sa, sb):
        wid = lax.axis_index("s") * n_cores + lax.axis_index("c")
        pltpu.sync_copy(idx_hbm.at[wid], idx_v)

        lanes = lax.iota(jnp.int32, 16)
        zeros = jnp.zeros((16,), jnp.int32)

        def gather(s, buf, sem):
            pltpu.async_copy(table_hbm.at[idx_v.at[s]], buf, sem)

        def gather_wait(buf, sem):
            pltpu.make_async_copy(table_hbm.at[idx_v.at[0]], buf, sem).wait()

        def transpose_scale(gbuf, tbuf):
            def rbody(r, c):
                col = zeros + r
                for kk in range(d // 16):
                    erow = lanes + 16 * kk
                    v = gbuf[r, pl.ds(16 * kk, 16)]
                    plsc.store_scatter(tbuf, [erow, col], v * scale)
                return c

            lax.fori_loop(0, _LB, rbody, 0)

        def store(s, tbuf, sem):
            for eh in range(d_hi):
                pltpu.async_copy(
                    tbuf.at[pl.ds(8 * eh, 8), pl.ds(0, _LB)],
                    out_hbm.at[s, eh, wid],
                    sem,
                )

        def store_wait(tbuf, sem):
            for eh in range(d_hi):
                pltpu.make_async_copy(
                    tbuf.at[pl.ds(8 * eh, 8), pl.ds(0, _LB)],
                    out_hbm.at[0, eh, wid],
                    sem,
                ).wait()

        gather(0, g_a, ga)

        def body(p, carry):
            s0 = 2 * p

            @pl.when(p > 0)
            def _():
                store_wait(t_a, sa)

            gather(s0 + 1, g_b, gb)
            gather_wait(g_a, ga)
            transpose_scale(g_a, t_a)
            store(s0, t_a, sa)

            @pl.when(p < n_seq // 2 - 1)
            def _():
                gather(s0 + 2, g_a, ga)

            @pl.when(p > 0)
            def _():
                store_wait(t_b, sb)

            gather_wait(g_b, gb)
            transpose_scale(g_b, t_b)
            store(s0 + 1, t_b, sb)
            return carry

        lax.fori_loop(0, n_seq // 2, body, 0)
        store_wait(t_a, sa)
        store_wait(t_b, sb)

    return k(table, idx)


def kernel(tokens, table):
    b0, b1 = tokens.shape
    vocab, d = table.shape
    n_workers = b0 // _LB
    t2 = _detile(table.T)
    table_lin = t2.reshape(t2.shape[0] * 2, d)
    half = _CB // 2
    tok = tokens.astype(jnp.int32)
    phys = (
        (tok // _CB) * _CB
        + (tok % half) * 2
        + (tok % _CB) // half
    )
    idx = phys.T.reshape(b1, n_workers, _LB).transpose(1, 0, 2)
    out5 = _emb_lookup(table_lin, idx, n_workers=n_workers, n_seq=b1)
    return out5.transpose(2, 4, 0, 1, 3).reshape(b0, b1, d)


# speedup vs baseline: 1.8927x; 1.1730x over previous
import functools
import math

import jax
import jax.numpy as jnp
from jax import lax
from jax.experimental import pallas as pl
from jax.experimental.pallas import tpu as pltpu
from jax.experimental.pallas import tpu_sc as plsc

_LB = 128
_TP = 129
_CB = 2048


def _detile_body(scale, x_ref, o_ref):
    x = x_ref[...]
    h = _CB // 2
    x2 = jnp.concatenate([x[:, :h], x[:, h:]], axis=0)
    o_ref[...] = jnp.transpose(x2) * scale


def _detile(table_t, scale):
    emb, vocab = table_t.shape
    grid = (vocab + _CB - 1) // _CB
    return pl.pallas_call(
        functools.partial(_detile_body, scale),
        grid=(grid,),
        in_specs=[pl.BlockSpec((emb, _CB), lambda i: (0, i))],
        out_specs=pl.BlockSpec((_CB // 2, 2 * emb), lambda i: (i, 0)),
        out_shape=jax.ShapeDtypeStruct((grid * _CB // 2, 2 * emb), jnp.float32),
    )(table_t)


@functools.partial(jax.jit, static_argnames=("n_workers", "n_seq"))
def _emb_lookup(table, idx, *, n_workers, n_seq):
    vocab, d = table.shape
    mesh = plsc.VectorSubcoreMesh(core_axis_name="c", subcore_axis_name="s")
    n_cores = plsc.get_sparse_core_info().num_cores
    d_hi = d // 8

    @functools.partial(
        pl.kernel,
        mesh=mesh,
        out_type=jax.ShapeDtypeStruct((n_seq, d_hi, n_workers, 8, _LB), jnp.float32),
        scratch_types=[
            pltpu.VMEM((n_seq, _LB), jnp.int32),
            pltpu.VMEM((_LB, d), jnp.float32),
            pltpu.VMEM((_LB, d), jnp.float32),
            pltpu.VMEM((d, _TP), jnp.float32),
            pltpu.VMEM((d, _TP), jnp.float32),
            pltpu.SemaphoreType.DMA,
            pltpu.SemaphoreType.DMA,
            pltpu.SemaphoreType.DMA,
            pltpu.SemaphoreType.DMA,
        ],
        compiler_params=pltpu.CompilerParams(
            use_tc_tiling_on_sc=False, needs_layout_passes=False
        ),
    )
    def k(table_hbm, idx_hbm, out_hbm, idx_v, g_a, g_b, t_a, t_b, ga, gb, sa, sb):
        wid = lax.axis_index("s") * n_cores + lax.axis_index("c")
        pltpu.sync_copy(idx_hbm.at[wid], idx_v)

        lanes = lax.iota(jnp.int32, 16)
        zeros = jnp.zeros((16,), jnp.int32)

        def gather(s, buf, sem):
            pltpu.async_copy(table_hbm.at[idx_v.at[s]], buf, sem)

        def gather_wait(buf, sem):
            pltpu.make_async_copy(table_hbm.at[idx_v.at[0]], buf, sem).wait()

        erows = [lanes + 16 * kk for kk in range(d // 16)]

        def transpose_scale(gbuf, tbuf):
            def rbody(r4, c):
                r0 = 4 * r4
                for dr in range(4):
                    col = zeros + (r0 + dr)
                    for kk in range(d // 16):
                        v = gbuf[r0 + dr, pl.ds(16 * kk, 16)]
                        plsc.store_scatter(tbuf, [erows[kk], col], v)
                return c

            lax.fori_loop(0, _LB // 4, rbody, 0)

        def store(s, tbuf, sem):
            for eh in range(d_hi):
                pltpu.async_copy(
                    tbuf.at[pl.ds(8 * eh, 8), pl.ds(0, _LB)],
                    out_hbm.at[s, eh, wid],
                    sem,
                )

        def store_wait(tbuf, sem):
            for eh in range(d_hi):
                pltpu.make_async_copy(
                    tbuf.at[pl.ds(8 * eh, 8), pl.ds(0, _LB)],
                    out_hbm.at[0, eh, wid],
                    sem,
                ).wait()

        gather(0, g_a, ga)

        def body(p, carry):
            s0 = 2 * p

            @pl.when(p > 0)
            def _():
                store_wait(t_a, sa)

            gather(s0 + 1, g_b, gb)
            gather_wait(g_a, ga)
            transpose_scale(g_a, t_a)
            store(s0, t_a, sa)

            @pl.when(p < n_seq // 2 - 1)
            def _():
                gather(s0 + 2, g_a, ga)

            @pl.when(p > 0)
            def _():
                store_wait(t_b, sb)

            gather_wait(g_b, gb)
            transpose_scale(g_b, t_b)
            store(s0 + 1, t_b, sb)
            return carry

        lax.fori_loop(0, n_seq // 2, body, 0)
        store_wait(t_a, sa)
        store_wait(t_b, sb)

    return k(table, idx)


def kernel(tokens, table):
    b0, b1 = tokens.shape
    vocab, d = table.shape
    n_workers = b0 // _LB
    t2 = _detile(table.T, math.sqrt(d))
    table_lin = t2.reshape(t2.shape[0] * 2, d)
    half = _CB // 2
    tok = tokens.astype(jnp.int32)
    phys = (
        (tok // _CB) * _CB
        + (tok % half) * 2
        + (tok % _CB) // half
    )
    idx = phys.T.reshape(b1, n_workers, _LB).transpose(1, 0, 2)
    out5 = _emb_lookup(table_lin, idx, n_workers=n_workers, n_seq=b1)
    return out5.transpose(2, 4, 0, 1, 3).reshape(b0, b1, d)


# speedup vs baseline: 2.2128x; 1.1691x over previous
import functools
import math

import jax
import jax.numpy as jnp
from jax import lax
from jax.experimental import pallas as pl
from jax.experimental.pallas import tpu as pltpu
from jax.experimental.pallas import tpu_sc as plsc

_LB = 128
_TP = 129
_CB = 4096


def _detile_body(scale, x_ref, o_ref):
    x = x_ref[...]
    h = _CB // 2
    x2 = jnp.concatenate([x[:, :h], x[:, h:]], axis=0)
    o_ref[...] = jnp.transpose(x2) * scale


def _detile(table_t, scale):
    emb, vocab = table_t.shape
    grid = (vocab + _CB - 1) // _CB
    return pl.pallas_call(
        functools.partial(_detile_body, scale),
        grid=(grid,),
        in_specs=[pl.BlockSpec((emb, _CB), lambda i: (0, i))],
        out_specs=pl.BlockSpec((_CB // 2, 2 * emb), lambda i: (i, 0)),
        out_shape=jax.ShapeDtypeStruct((grid * _CB // 2, 2 * emb), jnp.float32),
    )(table_t)


@functools.partial(jax.jit, static_argnames=("n_workers", "n_seq"))
def _emb_lookup(table, idx, *, n_workers, n_seq):
    vocab, d = table.shape
    mesh = plsc.VectorSubcoreMesh(core_axis_name="c", subcore_axis_name="s")
    n_cores = plsc.get_sparse_core_info().num_cores
    d_hi = d // 8

    @functools.partial(
        pl.kernel,
        mesh=mesh,
        out_type=jax.ShapeDtypeStruct((n_seq, d_hi, n_workers, 8, _LB), jnp.float32),
        scratch_types=[
            pltpu.VMEM((n_seq, _LB), jnp.int32),
            pltpu.VMEM((_LB, d), jnp.float32),
            pltpu.VMEM((_LB, d), jnp.float32),
            pltpu.VMEM((d, _TP), jnp.float32),
            pltpu.VMEM((d, _TP), jnp.float32),
            pltpu.SemaphoreType.DMA,
            pltpu.SemaphoreType.DMA,
            pltpu.SemaphoreType.DMA,
            pltpu.SemaphoreType.DMA,
        ],
        compiler_params=pltpu.CompilerParams(
            use_tc_tiling_on_sc=False, needs_layout_passes=False
        ),
    )
    def k(table_hbm, idx_hbm, out_hbm, idx_v, g_a, g_b, t_a, t_b, ga, gb, sa, sb):
        wid = lax.axis_index("s") * n_cores + lax.axis_index("c")
        pltpu.sync_copy(idx_hbm.at[wid], idx_v)

        lanes = lax.iota(jnp.int32, 16)
        zeros = jnp.zeros((16,), jnp.int32)

        def gather(s, buf, sem):
            pltpu.async_copy(table_hbm.at[idx_v.at[s]], buf, sem)

        def gather_wait(buf, sem):
            pltpu.make_async_copy(table_hbm.at[idx_v.at[0]], buf, sem).wait()

        erows = [lanes + 16 * kk for kk in range(d // 16)]

        def transpose_scale(gbuf, tbuf):
            def rbody(r4, c):
                r0 = 4 * r4
                for dr in range(4):
                    col = zeros + (r0 + dr)
                    for kk in range(d // 16):
                        v = gbuf[r0 + dr, pl.ds(16 * kk, 16)]
                        plsc.store_scatter(tbuf, [erows[kk], col], v)
                return c

            lax.fori_loop(0, _LB // 4, rbody, 0)

        def store(s, tbuf, sem):
            for eh in range(d_hi):
                pltpu.async_copy(
                    tbuf.at[pl.ds(8 * eh, 8), pl.ds(0, _LB)],
                    out_hbm.at[s, eh, wid],
                    sem,
                )

        def store_wait(tbuf, sem):
            for eh in range(d_hi):
                pltpu.make_async_copy(
                    tbuf.at[pl.ds(8 * eh, 8), pl.ds(0, _LB)],
                    out_hbm.at[0, eh, wid],
                    sem,
                ).wait()

        gather(0, g_a, ga)

        def body(p, carry):
            s0 = 2 * p

            @pl.when(p > 0)
            def _():
                store_wait(t_a, sa)

            gather(s0 + 1, g_b, gb)
            gather_wait(g_a, ga)
            transpose_scale(g_a, t_a)
            store(s0, t_a, sa)

            @pl.when(p < n_seq // 2 - 1)
            def _():
                gather(s0 + 2, g_a, ga)

            @pl.when(p > 0)
            def _():
                store_wait(t_b, sb)

            gather_wait(g_b, gb)
            transpose_scale(g_b, t_b)
            store(s0 + 1, t_b, sb)
            return carry

        lax.fori_loop(0, n_seq // 2, body, 0)
        store_wait(t_a, sa)
        store_wait(t_b, sb)

    return k(table, idx)


def kernel(tokens, table):
    b0, b1 = tokens.shape
    vocab, d = table.shape
    n_workers = b0 // _LB
    t2 = _detile(table.T, math.sqrt(d))
    table_lin = t2.reshape(t2.shape[0] * 2, d)
    half = _CB // 2
    tok = tokens.astype(jnp.int32)
    phys = (
        (tok // _CB) * _CB
        + (tok % half) * 2
        + (tok % _CB) // half
    )
    idx = phys.T.reshape(b1, n_workers, _LB).transpose(1, 0, 2)
    out5 = _emb_lookup(table_lin, idx, n_workers=n_workers, n_seq=b1)
    return out5.transpose(2, 4, 0, 1, 3).reshape(b0, b1, d)


# speedup vs baseline: 2.4808x; 1.1211x over previous
import functools
import math

import jax
import jax.numpy as jnp
from jax import lax
from jax.experimental import pallas as pl
from jax.experimental.pallas import tpu as pltpu
from jax.experimental.pallas import tpu_sc as plsc

_LB = 128
_TP = 129
_CB = 8192


def _detile_body(scale, x_ref, o_ref):
    x = x_ref[...]
    h = _CB // 2
    x2 = jnp.concatenate([x[:, :h], x[:, h:]], axis=0)
    o_ref[...] = jnp.transpose(x2) * scale


def _detile(table_t, scale):
    emb, vocab = table_t.shape
    grid = (vocab + _CB - 1) // _CB
    return pl.pallas_call(
        functools.partial(_detile_body, scale),
        grid=(grid,),
        in_specs=[pl.BlockSpec((emb, _CB), lambda i: (0, i))],
        out_specs=pl.BlockSpec((_CB // 2, 2 * emb), lambda i: (i, 0)),
        out_shape=jax.ShapeDtypeStruct((grid * _CB // 2, 2 * emb), jnp.float32),
    )(table_t)


@functools.partial(jax.jit, static_argnames=("n_workers", "n_seq"))
def _emb_lookup(table, idx, *, n_workers, n_seq):
    vocab, d = table.shape
    mesh = plsc.VectorSubcoreMesh(core_axis_name="c", subcore_axis_name="s")
    n_cores = plsc.get_sparse_core_info().num_cores
    d_hi = d // 8

    @functools.partial(
        pl.kernel,
        mesh=mesh,
        out_type=jax.ShapeDtypeStruct((n_seq, d_hi, n_workers, 8, _LB), jnp.float32),
        scratch_types=[
            pltpu.VMEM((n_seq, _LB), jnp.int32),
            pltpu.VMEM((_LB, d), jnp.float32),
            pltpu.VMEM((_LB, d), jnp.float32),
            pltpu.VMEM((d, _TP), jnp.float32),
            pltpu.VMEM((d, _TP), jnp.float32),
            pltpu.SemaphoreType.DMA,
            pltpu.SemaphoreType.DMA,
            pltpu.SemaphoreType.DMA,
            pltpu.SemaphoreType.DMA,
        ],
        compiler_params=pltpu.CompilerParams(
            use_tc_tiling_on_sc=False, needs_layout_passes=False
        ),
    )
    def k(table_hbm, idx_hbm, out_hbm, idx_v, g_a, g_b, t_a, t_b, ga, gb, sa, sb):
        wid = lax.axis_index("s") * n_cores + lax.axis_index("c")
        pltpu.sync_copy(idx_hbm.at[wid], idx_v)

        lanes = lax.iota(jnp.int32, 16)
        zeros = jnp.zeros((16,), jnp.int32)

        def gather(s, buf, sem):
            pltpu.async_copy(table_hbm.at[idx_v.at[s]], buf, sem)

        def gather_wait(buf, sem):
            pltpu.make_async_copy(table_hbm.at[idx_v.at[0]], buf, sem).wait()

        erows = [lanes + 16 * kk for kk in range(d // 16)]

        def transpose_scale(gbuf, tbuf):
            def rbody(r4, c):
                r0 = 4 * r4
                for dr in range(4):
                    col = zeros + (r0 + dr)
                    for kk in range(d // 16):
                        v = gbuf[r0 + dr, pl.ds(16 * kk, 16)]
                        plsc.store_scatter(tbuf, [erows[kk], col], v)
                return c

            lax.fori_loop(0, _LB // 4, rbody, 0)

        def store(s, tbuf, sem):
            for eh in range(d_hi):
                pltpu.async_copy(
                    tbuf.at[pl.ds(8 * eh, 8), pl.ds(0, _LB)],
                    out_hbm.at[s, eh, wid],
                    sem,
                )

        def store_wait(tbuf, sem):
            for eh in range(d_hi):
                pltpu.make_async_copy(
                    tbuf.at[pl.ds(8 * eh, 8), pl.ds(0, _LB)],
                    out_hbm.at[0, eh, wid],
                    sem,
                ).wait()

        gather(0, g_a, ga)

        def body(p, carry):
            s0 = 2 * p

            @pl.when(p > 0)
            def _():
                store_wait(t_a, sa)

            gather(s0 + 1, g_b, gb)
            gather_wait(g_a, ga)
            transpose_scale(g_a, t_a)
            store(s0, t_a, sa)

            @pl.when(p < n_seq // 2 - 1)
            def _():
                gather(s0 + 2, g_a, ga)

            @pl.when(p > 0)
            def _():
                store_wait(t_b, sb)

            gather_wait(g_b, gb)
            transpose_scale(g_b, t_b)
            store(s0 + 1, t_b, sb)
            return carry

        lax.fori_loop(0, n_seq // 2, body, 0)
        store_wait(t_a, sa)
        store_wait(t_b, sb)

    return k(table, idx)


def kernel(tokens, table):
    b0, b1 = tokens.shape
    vocab, d = table.shape
    n_workers = b0 // _LB
    t2 = _detile(table.T, math.sqrt(d))
    table_lin = t2.reshape(t2.shape[0] * 2, d)
    half = _CB // 2
    tok = tokens.astype(jnp.int32)
    phys = (
        (tok // _CB) * _CB
        + (tok % half) * 2
        + (tok % _CB) // half
    )
    idx = phys.T.reshape(b1, n_workers, _LB).transpose(1, 0, 2)
    out5 = _emb_lookup(table_lin, idx, n_workers=n_workers, n_seq=b1)
    return out5.transpose(2, 4, 0, 1, 3).reshape(b0, b1, d)


# speedup vs baseline: 2.6020x; 1.0489x over previous
import functools
import math

import jax
import jax.numpy as jnp
from jax import lax
from jax.experimental import pallas as pl
from jax.experimental.pallas import tpu as pltpu
from jax.experimental.pallas import tpu_sc as plsc

_LB = 128
_TP = 129
_CB = 16384


def _detile_body(scale, x_ref, o_ref):
    x = x_ref[...]
    h = _CB // 2
    x2 = jnp.concatenate([x[:, :h], x[:, h:]], axis=0)
    o_ref[...] = jnp.transpose(x2) * scale


def _detile(table_t, scale):
    emb, vocab = table_t.shape
    grid = (vocab + _CB - 1) // _CB
    return pl.pallas_call(
        functools.partial(_detile_body, scale),
        grid=(grid,),
        in_specs=[pl.BlockSpec((emb, _CB), lambda i: (0, i))],
        out_specs=pl.BlockSpec((_CB // 2, 2 * emb), lambda i: (i, 0)),
        out_shape=jax.ShapeDtypeStruct((grid * _CB // 2, 2 * emb), jnp.float32),
    )(table_t)


@functools.partial(jax.jit, static_argnames=("n_workers", "n_seq"))
def _emb_lookup(table, idx, *, n_workers, n_seq):
    vocab, d = table.shape
    mesh = plsc.VectorSubcoreMesh(core_axis_name="c", subcore_axis_name="s")
    n_cores = plsc.get_sparse_core_info().num_cores
    d_hi = d // 8

    @functools.partial(
        pl.kernel,
        mesh=mesh,
        out_type=jax.ShapeDtypeStruct((n_seq, d_hi, n_workers, 8, _LB), jnp.float32),
        scratch_types=[
            pltpu.VMEM((n_seq, _LB), jnp.int32),
            pltpu.VMEM((_LB, d), jnp.float32),
            pltpu.VMEM((_LB, d), jnp.float32),
            pltpu.VMEM((d, _TP), jnp.float32),
            pltpu.VMEM((d, _TP), jnp.float32),
            pltpu.SemaphoreType.DMA,
            pltpu.SemaphoreType.DMA,
            pltpu.SemaphoreType.DMA,
            pltpu.SemaphoreType.DMA,
        ],
        compiler_params=pltpu.CompilerParams(
            use_tc_tiling_on_sc=False, needs_layout_passes=False
        ),
    )
    def k(table_hbm, idx_hbm, out_hbm, idx_v, g_a, g_b, t_a, t_b, ga, gb, sa, sb):
        wid = lax.axis_index("s") * n_cores + lax.axis_index("c")
        pltpu.sync_copy(idx_hbm.at[wid], idx_v)

        lanes = lax.iota(jnp.int32, 16)
        zeros = jnp.zeros((16,), jnp.int32)

        def gather(s, buf, sem):
            pltpu.async_copy(table_hbm.at[idx_v.at[s]], buf, sem)

        def gather_wait(buf, sem):
            pltpu.make_async_copy(table_hbm.at[idx_v.at[0]], buf, sem).wait()

        erows = [lanes + 16 * kk for kk in range(d // 16)]

        def transpose_scale(gbuf, tbuf):
            def rbody(r4, c):
                r0 = 4 * r4
                for dr in range(4):
                    col = zeros + (r0 + dr)
                    for kk in range(d // 16):
                        v = gbuf[r0 + dr, pl.ds(16 * kk, 16)]
                        plsc.store_scatter(tbuf, [erows[kk], col], v)
                return c

            lax.fori_loop(0, _LB // 4, rbody, 0)

        def store(s, tbuf, sem):
            for eh in range(d_hi):
                pltpu.async_copy(
                    tbuf.at[pl.ds(8 * eh, 8), pl.ds(0, _LB)],
                    out_hbm.at[s, eh, wid],
                    sem,
                )

        def store_wait(tbuf, sem):
            for eh in range(d_hi):
                pltpu.make_async_copy(
                    tbuf.at[pl.ds(8 * eh, 8), pl.ds(0, _LB)],
                    out_hbm.at[0, eh, wid],
                    sem,
                ).wait()

        gather(0, g_a, ga)

        def body(p, carry):
            s0 = 2 * p

            @pl.when(p > 0)
            def _():
                store_wait(t_a, sa)

            gather(s0 + 1, g_b, gb)
            gather_wait(g_a, ga)
            transpose_scale(g_a, t_a)
            store(s0, t_a, sa)

            @pl.when(p < n_seq // 2 - 1)
            def _():
                gather(s0 + 2, g_a, ga)

            @pl.when(p > 0)
            def _():
                store_wait(t_b, sb)

            gather_wait(g_b, gb)
            transpose_scale(g_b, t_b)
            store(s0 + 1, t_b, sb)
            return carry

        lax.fori_loop(0, n_seq // 2, body, 0)
        store_wait(t_a, sa)
        store_wait(t_b, sb)

    return k(table, idx)


def kernel(tokens, table):
    b0, b1 = tokens.shape
    vocab, d = table.shape
    n_workers = b0 // _LB
    t2 = _detile(table.T, math.sqrt(d))
    table_lin = t2.reshape(t2.shape[0] * 2, d)
    half = _CB // 2
    tok = tokens.astype(jnp.int32)
    phys = (
        (tok // _CB) * _CB
        + (tok % half) * 2
        + (tok % _CB) // half
    )
    idx = phys.T.reshape(b1, n_workers, _LB).transpose(1, 0, 2)
    out5 = _emb_lookup(table_lin, idx, n_workers=n_workers, n_seq=b1)
    return out5.transpose(2, 4, 0, 1, 3).reshape(b0, b1, d)


# speedup vs baseline: 2.6342x; 1.0124x over previous
import functools
import math

import jax
import jax.numpy as jnp
from jax import lax
from jax.experimental import pallas as pl
from jax.experimental.pallas import tpu as pltpu
from jax.experimental.pallas import tpu_sc as plsc

_LB = 128
_TP = 129
_CB = 32768


def _detile_body(scale, x_ref, o_ref):
    x = x_ref[...]
    h = _CB // 2
    x2 = jnp.concatenate([x[:, :h], x[:, h:]], axis=0)
    o_ref[...] = jnp.transpose(x2) * scale


def _detile(table_t, scale):
    emb, vocab = table_t.shape
    grid = (vocab + _CB - 1) // _CB
    return pl.pallas_call(
        functools.partial(_detile_body, scale),
        grid=(grid,),
        in_specs=[pl.BlockSpec((emb, _CB), lambda i: (0, i))],
        out_specs=pl.BlockSpec((_CB // 2, 2 * emb), lambda i: (i, 0)),
        out_shape=jax.ShapeDtypeStruct((grid * _CB // 2, 2 * emb), jnp.float32),
    )(table_t)


@functools.partial(jax.jit, static_argnames=("n_workers", "n_seq"))
def _emb_lookup(table, idx, *, n_workers, n_seq):
    vocab, d = table.shape
    mesh = plsc.VectorSubcoreMesh(core_axis_name="c", subcore_axis_name="s")
    n_cores = plsc.get_sparse_core_info().num_cores
    d_hi = d // 8

    @functools.partial(
        pl.kernel,
        mesh=mesh,
        out_type=jax.ShapeDtypeStruct((n_seq, d_hi, n_workers, 8, _LB), jnp.float32),
        scratch_types=[
            pltpu.VMEM((n_seq, _LB), jnp.int32),
            pltpu.VMEM((_LB, d), jnp.float32),
            pltpu.VMEM((_LB, d), jnp.float32),
            pltpu.VMEM((d, _TP), jnp.float32),
            pltpu.VMEM((d, _TP), jnp.float32),
            pltpu.SemaphoreType.DMA,
            pltpu.SemaphoreType.DMA,
            pltpu.SemaphoreType.DMA,
            pltpu.SemaphoreType.DMA,
        ],
        compiler_params=pltpu.CompilerParams(
            use_tc_tiling_on_sc=False, needs_layout_passes=False
        ),
    )
    def k(table_hbm, idx_hbm, out_hbm, idx_v, g_a, g_b, t_a, t_b, ga, gb, sa, sb):
        wid = lax.axis_index("s") * n_cores + lax.axis_index("c")
        pltpu.sync_copy(idx_hbm.at[wid], idx_v)

        lanes = lax.iota(jnp.int32, 16)
        zeros = jnp.zeros((16,), jnp.int32)

        def gather(s, buf, sem):
            pltpu.async_copy(table_hbm.at[idx_v.at[s]], buf, sem)

        def gather_wait(buf, sem):
            pltpu.make_async_copy(table_hbm.at[idx_v.at[0]], buf, sem).wait()

        erows = [lanes + 16 * kk for kk in range(d // 16)]

        def transpose_scale(gbuf, tbuf):
            def rbody(r4, c):
                r0 = 4 * r4
                for dr in range(4):
                    col = zeros + (r0 + dr)
                    for kk in range(d // 16):
                        v = gbuf[r0 + dr, pl.ds(16 * kk, 16)]
                        plsc.store_scatter(tbuf, [erows[kk], col], v)
                return c

            lax.fori_loop(0, _LB // 4, rbody, 0)

        def store(s, tbuf, sem):
            for eh in range(d_hi):
                pltpu.async_copy(
                    tbuf.at[pl.ds(8 * eh, 8), pl.ds(0, _LB)],
                    out_hbm.at[s, eh, wid],
                    sem,
                )

        def store_wait(tbuf, sem):
            for eh in range(d_hi):
                pltpu.make_async_copy(
                    tbuf.at[pl.ds(8 * eh, 8), pl.ds(0, _LB)],
                    out_hbm.at[0, eh, wid],
                    sem,
                ).wait()

        gather(0, g_a, ga)

        def body(p, carry):
            s0 = 2 * p

            @pl.when(p > 0)
            def _():
                store_wait(t_a, sa)

            gather(s0 + 1, g_b, gb)
            gather_wait(g_a, ga)
            transpose_scale(g_a, t_a)
            store(s0, t_a, sa)

            @pl.when(p < n_seq // 2 - 1)
            def _():
                gather(s0 + 2, g_a, ga)

            @pl.when(p > 0)
            def _():
                store_wait(t_b, sb)

            gather_wait(g_b, gb)
            transpose_scale(g_b, t_b)
            store(s0 + 1, t_b, sb)
            return carry

        lax.fori_loop(0, n_seq // 2, body, 0)
        store_wait(t_a, sa)
        store_wait(t_b, sb)

    return k(table, idx)


def kernel(tokens, table):
    b0, b1 = tokens.shape
    vocab, d = table.shape
    n_workers = b0 // _LB
    t2 = _detile(table.T, math.sqrt(d))
    table_lin = t2.reshape(t2.shape[0] * 2, d)
    half = _CB // 2
    tok = tokens.astype(jnp.int32)
    phys = (
        (tok // _CB) * _CB
        + (tok % half) * 2
        + (tok % _CB) // half
    )
    idx = phys.T.reshape(b1, n_workers, _LB).transpose(1, 0, 2)
    out5 = _emb_lookup(table_lin, idx, n_workers=n_workers, n_seq=b1)
    return out5.transpose(2, 4, 0, 1, 3).reshape(b0, b1, d)
